# baseline (speedup 1.0000x reference)
import functools

import jax
import jax.numpy as jnp
from jax import lax
from jax.experimental import pallas as pl
from jax.experimental.pallas import tpu as pltpu
from jax.experimental.pallas import tpu_sc as plsc

D = 32
NC = 2
NS = 16
NW = NC * NS
CHUNK = 128
PER_W = 512
NCHUNK = PER_W // CHUNK


def _make_gather(n_rows: int, n_cols: int):
    assert n_rows == NW * PER_W
    assert (n_cols - 2) % 2 == 0 and n_cols >= 4

    mesh = plsc.VectorSubcoreMesh(core_axis_name="c", subcore_axis_name="s")

    @functools.partial(
        pl.kernel,
        mesh=mesh,
        compiler_params=pltpu.CompilerParams(
            use_tc_tiling_on_sc=False, needs_layout_passes=False),
        out_type=jax.ShapeDtypeStruct(
            (n_cols, D // 8, n_rows // 128, 8, 128), jnp.float32),
        scratch_types=[
            pltpu.VMEM((PER_W, n_cols), jnp.int32),
            pltpu.VMEM((n_cols, PER_W), jnp.int32),
            pltpu.VMEM((PER_W, D), jnp.float32),
            pltpu.VMEM((PER_W, D), jnp.float32),
            pltpu.VMEM((D // 8, PER_W // 128, 8, 128), jnp.float32),
            pltpu.VMEM((D // 8, PER_W // 128, 8, 128), jnp.float32),
            pltpu.SemaphoreType.DMA,
            pltpu.SemaphoreType.DMA,
            pltpu.SemaphoreType.DMA,
            pltpu.SemaphoreType.DMA,
        ],
    )
    def gather(loc_hbm, table_hbm, out_hbm, idxr, idx_v, rows0, rows1,
               tr0, tr1, gsem0, gsem1, wsem0, wsem1):
        wid = lax.axis_index("s") * NC + lax.axis_index("c")
        i0 = wid * PER_W
        rows = (rows0, rows1)
        tr = (tr0, tr1)
        gsem = (gsem0, gsem1)
        wsem = (wsem0, wsem1)

        pltpu.sync_copy(loc_hbm.at[pl.ds(i0, PER_W)], idxr)

        @plsc.parallel_loop(0, PER_W // 16, unroll=2)
        def _(ib):
            lanes = lax.iota(jnp.int32, 16)
            ridx = ib * 16 + lanes
            for j in range(n_cols):
                jv = lax.rem(j + lanes, n_cols)
                v = plsc.load_gather(idxr, [ridx, jv])
                plsc.store_scatter(idx_v, [jv, ridx], v)

        def issue_gathers(j, a):
            for c in range(NCHUNK):
                pltpu.async_copy(
                    table_hbm.at[idx_v.at[j, pl.ds(c * CHUNK, CHUNK)]],
                    rows[a].at[pl.ds(c * CHUNK, CHUNK)],
                    gsem[a],
                )

        def wait_gathers(a):
            pltpu.make_async_copy(
                table_hbm.at[pl.ds(0, PER_W)], rows[a], gsem[a]).wait()

        def transpose(a):
            @plsc.parallel_loop(0, PER_W // 16, unroll=2)
            def _(ib):
                lanes = lax.iota(jnp.int32, 16)
                ridx = ib * 16 + lanes
                itv = ridx >> 7
                i128 = ridx & 127
                for d in range(D):
                    kvec = (d + lanes) & (D - 1)
                    v = plsc.load_gather(rows[a], [ridx, kvec])
                    plsc.store_scatter(
                        tr[a], [kvec >> 3, itv, kvec & 7, i128], v)

        it0 = wid * (PER_W // 128)

        def issue_write(j, a):
            pltpu.async_copy(
                tr[a], out_hbm.at[j, :, pl.ds(it0, PER_W // 128)], wsem[a])

        def wait_write(a):
            pltpu.make_async_copy(
                tr[a], out_hbm.at[0, :, pl.ds(0, PER_W // 128)], wsem[a]).wait()

        issue_gathers(0, 0)
        wait_gathers(0)
        issue_gathers(1, 1)
        transpose(0)
        issue_write(0, 0)
        pltpu.async_copy(
            tr0, out_hbm.at[0, :, pl.ds(it0, PER_W // 128)], wsem1)

        @pl.loop(1, n_cols - 1, step=2)
        def _(g0):
            for a in (1, 0):
                j = g0 if a == 1 else g0 + 1
                wait_gathers(a)
                issue_gathers(j + 1, 1 - a)
                wait_write(a)
                transpose(a)
                issue_write(j, a)

        wait_gathers(1)
        wait_write(1)
        transpose(1)
        issue_write(n_cols - 1, 1)
        wait_write(0)
        wait_write(1)

    return gather


@jax.jit
def _run(location, embedding):
    n_rows, n_cols = location.shape
    out5 = _make_gather(n_rows, n_cols)(location.astype(jnp.int32),
                                        embedding)
    out = jnp.transpose(out5, (2, 4, 0, 1, 3))
    return out.reshape(n_rows, n_cols, D)


def kernel(location, embedding):
    return _run(location, embedding)


# speedup vs baseline: 1.1213x; 1.1213x over previous
import functools

import jax
import jax.numpy as jnp
from jax import lax
from jax.experimental import pallas as pl
from jax.experimental.pallas import tpu as pltpu
from jax.experimental.pallas import tpu_sc as plsc

D = 32
NC = 2
NS = 16
NW = NC * NS
CHUNK = 128
PER_W = 512
NCHUNK = PER_W // CHUNK


def _make_gather(n_rows: int, n_cols: int):
    assert n_rows == NW * PER_W
    assert (n_cols - 2) % 2 == 0 and n_cols >= 4

    mesh = plsc.VectorSubcoreMesh(core_axis_name="c", subcore_axis_name="s")

    @functools.partial(
        pl.kernel,
        mesh=mesh,
        compiler_params=pltpu.CompilerParams(
            use_tc_tiling_on_sc=False, needs_layout_passes=False),
        out_type=jax.ShapeDtypeStruct(
            (n_cols, D // 8, n_rows // 128, 8, 128), jnp.float32),
        scratch_types=[
            pltpu.VMEM((n_cols, PER_W), jnp.int32),
            pltpu.VMEM((PER_W, D), jnp.float32),
            pltpu.VMEM((PER_W, D), jnp.float32),
            pltpu.VMEM((D // 8, PER_W // 128, 8, 128), jnp.float32),
            pltpu.VMEM((D // 8, PER_W // 128, 8, 128), jnp.float32),
            pltpu.SemaphoreType.DMA,
            pltpu.SemaphoreType.DMA,
            pltpu.SemaphoreType.DMA,
            pltpu.SemaphoreType.DMA,
        ],
    )
    def gather(loc_hbm, table_hbm, out_hbm, idx_v, rows0, rows1, tr0, tr1,
               gsem0, gsem1, wsem0, wsem1):
        wid = lax.axis_index("s") * NC + lax.axis_index("c")
        i0 = wid * PER_W
        rows = (rows0, rows1)
        tr = (tr0, tr1)
        gsem = (gsem0, gsem1)
        wsem = (wsem0, wsem1)

        pltpu.sync_copy(loc_hbm.at[:, pl.ds(i0, PER_W)], idx_v)

        def issue_gathers(j, a):
            for c in range(NCHUNK):
                pltpu.async_copy(
                    table_hbm.at[idx_v.at[j, pl.ds(c * CHUNK, CHUNK)]],
                    rows[a].at[pl.ds(c * CHUNK, CHUNK)],
                    gsem[a],
                )

        def wait_gathers(a):
            pltpu.make_async_copy(
                table_hbm.at[pl.ds(0, PER_W)], rows[a], gsem[a]).wait()

        def transpose(a):
            @plsc.parallel_loop(0, PER_W // 16, unroll=2)
            def _(ib):
                lanes = lax.iota(jnp.int32, 16)
                ridx = ib * 16 + lanes
                itv = ridx >> 7
                i128 = ridx & 127
                for d in range(D):
                    kvec = (d + lanes) & (D - 1)
                    v = plsc.load_gather(rows[a], [ridx, kvec])
                    plsc.store_scatter(
                        tr[a], [kvec >> 3, itv, kvec & 7, i128], v)

        it0 = wid * (PER_W // 128)

        def issue_write(j, a):
            pltpu.async_copy(
                tr[a], out_hbm.at[j, :, pl.ds(it0, PER_W // 128)], wsem[a])

        def wait_write(a):
            pltpu.make_async_copy(
                tr[a], out_hbm.at[0, :, pl.ds(0, PER_W // 128)], wsem[a]).wait()

        issue_gathers(0, 0)
        wait_gathers(0)
        issue_gathers(1, 1)
        transpose(0)
        issue_write(0, 0)
        pltpu.async_copy(
            tr0, out_hbm.at[0, :, pl.ds(it0, PER_W // 128)], wsem1)

        @pl.loop(1, n_cols - 1, step=2)
        def _(g0):
            for a in (1, 0):
                j = g0 if a == 1 else g0 + 1
                wait_gathers(a)
                issue_gathers(j + 1, 1 - a)
                wait_write(a)
                transpose(a)
                issue_write(j, a)

        wait_gathers(1)
        wait_write(1)
        transpose(1)
        issue_write(n_cols - 1, 1)
        wait_write(0)
        wait_write(1)

    return gather


@jax.jit
def _run(location, embedding):
    n_rows, n_cols = location.shape
    loc_t = location.T.astype(jnp.int32)
    out5 = _make_gather(n_rows, n_cols)(loc_t, embedding)
    out = jnp.transpose(out5, (2, 4, 0, 1, 3))
    return out.reshape(n_rows, n_cols, D)


def kernel(location, embedding):
    return _run(location, embedding)
